# baseline (speedup 1.0000x reference)
import functools

import jax
import jax.numpy as jnp
from jax.experimental import pallas as pl
from jax.experimental.pallas import tpu as pltpu

CONF_THRES = 0.25
IOU_THRES = 0.45
MAX_DET = 300
MAX_WH = 4096.0
IMG = 640.0

N = 20000
NPAD = 20480
ROWS = 160
LANES = 128
SELR = 3

NEG_INF = float("-inf")
BIG = 2**30

DEC_CHUNK = 2048


def _decode_body(x_ref, s_ref, x1o_ref, y1o_ref, x2o_ref, y2o_ref, cls_ref):
    xb = x_ref[0]
    cx = jax.nn.sigmoid(xb[0:1, :]) * IMG
    cy = jax.nn.sigmoid(xb[1:2, :]) * IMG
    hw = (jax.nn.sigmoid(xb[2:3, :]) * IMG) * 0.5
    hh = (jax.nn.sigmoid(xb[3:4, :]) * IMG) * 0.5
    obj = jax.nn.sigmoid(xb[4:5, :])
    cls_p = jax.nn.sigmoid(xb[5:85, :])
    conf = obj * cls_p
    score = jnp.max(conf, axis=0, keepdims=True)
    ii = jax.lax.broadcasted_iota(jnp.int32, conf.shape, 0)
    cls = jnp.min(jnp.where(conf == score, ii, BIG),
                  axis=0, keepdims=True)
    clsf = cls.astype(jnp.float32)

    off = clsf * MAX_WH
    s_ref[0] = jnp.where(score > CONF_THRES, score, NEG_INF)
    x1o_ref[0] = (cx - hw) + off
    y1o_ref[0] = (cy - hh) + off
    x2o_ref[0] = (cx + hw) + off
    y2o_ref[0] = (cy + hh) + off
    cls_ref[0] = clsf


def _nms_body(s_ref, x1o_ref, y1o_ref, x2o_ref, y2o_ref, cls_ref, o_ref,
              sx1_ref, sy1_ref, sx2_ref, sy2_ref, sa_ref):
    B = s_ref.shape[0]
    flat = (jax.lax.broadcasted_iota(jnp.int32, (ROWS, LANES), 0) * LANES
            + jax.lax.broadcasted_iota(jnp.int32, (ROWS, LANES), 1))
    lane = jax.lax.broadcasted_iota(jnp.int32, (1, LANES), 1)
    lane6 = jax.lax.broadcasted_iota(jnp.int32, (1, 6), 1)
    selrow = jax.lax.broadcasted_iota(jnp.int32, (SELR, LANES), 0)
    sellane = jax.lax.broadcasted_iota(jnp.int32, (SELR, LANES), 1)

    o_ref[...] = jnp.zeros((B, MAX_DET, 6), jnp.float32)
    for b in range(B):
        sx1_ref[b] = jnp.full((SELR, LANES), -1e9)
        sy1_ref[b] = jnp.full((SELR, LANES), -1e9)
        sx2_ref[b] = jnp.full((SELR, LANES), -1e9)
        sy2_ref[b] = jnp.full((SELR, LANES), -1e9)
        sa_ref[b] = jnp.zeros((SELR, LANES), jnp.float32)

    def one_batch(b, state):
        v, ilast, cnt, alive = state
        s = s_ref[b]
        mask = (s < v) | ((s == v) & (flat > ilast))
        sm = jnp.where(mask, s, NEG_INF)
        m = jnp.max(sm)
        i = jnp.min(jnp.where(sm == m, flat, BIG))
        found = m > NEG_INF
        i_safe = jnp.where(found, i, 0)
        r = i_safe // LANES
        l = i_safe % LANES

        def ext(ref):
            row = ref[b, pl.ds(r, 1), :]
            return jnp.max(jnp.where(lane == l, row, NEG_INF))

        bx1o = ext(x1o_ref)
        by1o = ext(y1o_ref)
        bx2o = ext(x2o_ref)
        by2o = ext(y2o_ref)
        bcls = ext(cls_ref)
        a2c = (bx2o - bx1o) * (by2o - by1o)

        ltx = jnp.maximum(sx1_ref[b], bx1o)
        lty = jnp.maximum(sy1_ref[b], by1o)
        rbx = jnp.minimum(sx2_ref[b], bx2o)
        rby = jnp.minimum(sy2_ref[b], by2o)
        w = jnp.maximum(rbx - ltx, 0.0)
        h = jnp.maximum(rby - lty, 0.0)
        inter = w * h
        iou = inter / (sa_ref[b] + a2c - inter + 1e-7)
        supped = jnp.max(jnp.where(iou > IOU_THRES, 1.0, 0.0)) > 0.0
        accept = found & (~supped) & (cnt < MAX_DET)

        off = bcls * MAX_WH
        det = jnp.where(
            lane6 == 0, bx1o - off,
            jnp.where(lane6 == 1, by1o - off,
                      jnp.where(lane6 == 2, bx2o - off,
                                jnp.where(lane6 == 3, by2o - off,
                                          jnp.where(lane6 == 4, m, bcls)))))

        @pl.when(accept)
        def _():
            o_ref[b, pl.ds(cnt, 1), :] = det
            onehot = (selrow == cnt // LANES) & (sellane == cnt % LANES)
            sx1_ref[b] = jnp.where(onehot, bx1o, sx1_ref[b])
            sy1_ref[b] = jnp.where(onehot, by1o, sy1_ref[b])
            sx2_ref[b] = jnp.where(onehot, bx2o, sx2_ref[b])
            sy2_ref[b] = jnp.where(onehot, by2o, sy2_ref[b])
            sa_ref[b] = jnp.where(onehot, a2c, sa_ref[b])

        cnt2 = cnt + jnp.where(accept, 1, 0)
        v2 = jnp.where(found, m, v)
        i2 = jnp.where(found, i, ilast)
        alive2 = alive & found & (cnt2 < MAX_DET)
        return (v2, i2, cnt2, alive2)

    def cond(states):
        a = states[0][3]
        for st in states[1:]:
            a = a | st[3]
        return a

    def body(states):
        return tuple(
            jax.lax.cond(st[3], lambda s_=st, b_=b: one_batch(b_, s_),
                         lambda s_=st: s_)
            for b, st in enumerate(states)
        )

    init = tuple((jnp.float32(jnp.inf), jnp.int32(-1), jnp.int32(0),
                  jnp.bool_(True)) for _ in range(B))
    jax.lax.while_loop(cond, body, init)


@functools.partial(jax.jit, static_argnames=("interpret",))
def kernel(x, interpret=False):
    B = x.shape[0]
    xt = jnp.pad(x.transpose(0, 2, 1), ((0, 0), (0, 0), (0, NPAD - N)),
                 constant_values=-100.0)
    plane = jax.ShapeDtypeStruct((B, 1, NPAD), jnp.float32)
    planes = pl.pallas_call(
        _decode_body,
        grid=(B, NPAD // DEC_CHUNK),
        in_specs=[pl.BlockSpec((1, 85, DEC_CHUNK), lambda b, c: (b, 0, c))],
        out_specs=[pl.BlockSpec((1, 1, DEC_CHUNK), lambda b, c: (b, 0, c))] * 6,
        out_shape=[plane] * 6,
        interpret=interpret,
    )(xt)
    planes2d = [p.reshape(B, ROWS, LANES) for p in planes]
    dets = pl.pallas_call(
        _nms_body,
        in_specs=[pl.BlockSpec((B, ROWS, LANES), lambda: (0, 0, 0))] * 6,
        out_specs=pl.BlockSpec((B, MAX_DET, 6), lambda: (0, 0, 0)),
        out_shape=jax.ShapeDtypeStruct((B, MAX_DET, 6), jnp.float32),
        scratch_shapes=[pltpu.VMEM((B, SELR, LANES), jnp.float32)] * 5,
        interpret=interpret,
    )(*planes2d)
    return dets


# speedup vs baseline: 1.0014x; 1.0014x over previous
import functools

import jax
import jax.numpy as jnp
from jax.experimental import pallas as pl
from jax.experimental.pallas import tpu as pltpu

CONF_THRES = 0.25
IOU_THRES = 0.45
MAX_DET = 300
MAX_WH = 4096.0
IMG = 640.0

N = 20000
NPAD = 20480
ROWS = 160
LANES = 128
SELR = 3

NEG_INF = float("-inf")
BIG = 2**30

DEC_CHUNK = 2048


def _decode_body(x_ref, s_ref, x1o_ref, y1o_ref, x2o_ref, y2o_ref, cls_ref):
    xb = x_ref[0]
    cx = jax.nn.sigmoid(xb[0:1, :]) * IMG
    cy = jax.nn.sigmoid(xb[1:2, :]) * IMG
    hw = (jax.nn.sigmoid(xb[2:3, :]) * IMG) * 0.5
    hh = (jax.nn.sigmoid(xb[3:4, :]) * IMG) * 0.5
    obj = jax.nn.sigmoid(xb[4:5, :])
    cls_p = jax.nn.sigmoid(xb[5:85, :])
    conf = obj * cls_p
    score = jnp.max(conf, axis=0, keepdims=True)
    ii = jax.lax.broadcasted_iota(jnp.int32, conf.shape, 0)
    cls = jnp.min(jnp.where(conf == score, ii, BIG),
                  axis=0, keepdims=True)
    clsf = cls.astype(jnp.float32)

    off = clsf * MAX_WH
    s_ref[0] = jnp.where(score > CONF_THRES, score, NEG_INF)
    x1o_ref[0] = (cx - hw) + off
    y1o_ref[0] = (cy - hh) + off
    x2o_ref[0] = (cx + hw) + off
    y2o_ref[0] = (cy + hh) + off
    cls_ref[0] = clsf


def _nms_body(s_ref, x1o_ref, y1o_ref, x2o_ref, y2o_ref, cls_ref, o_ref,
              sx1_ref, sy1_ref, sx2_ref, sy2_ref, sa_ref):
    B = s_ref.shape[0]
    flat = (jax.lax.broadcasted_iota(jnp.int32, (ROWS, LANES), 0) * LANES
            + jax.lax.broadcasted_iota(jnp.int32, (ROWS, LANES), 1))
    lane = jax.lax.broadcasted_iota(jnp.int32, (1, LANES), 1)
    lane6 = jax.lax.broadcasted_iota(jnp.int32, (1, 6), 1)
    selrow = jax.lax.broadcasted_iota(jnp.int32, (SELR, LANES), 0)
    sellane = jax.lax.broadcasted_iota(jnp.int32, (SELR, LANES), 1)

    o_ref[...] = jnp.zeros((B, MAX_DET, 6), jnp.float32)
    for b in range(B):
        sx1_ref[b] = jnp.full((SELR, LANES), -1e9)
        sy1_ref[b] = jnp.full((SELR, LANES), -1e9)
        sx2_ref[b] = jnp.full((SELR, LANES), -1e9)
        sy2_ref[b] = jnp.full((SELR, LANES), -1e9)
        sa_ref[b] = jnp.zeros((SELR, LANES), jnp.float32)

    def one_batch(b, state):
        v, ilast, cnt, alive = state
        s = s_ref[b]
        mask = (s < v) | ((s == v) & (flat > ilast))
        sm = jnp.where(mask, s, NEG_INF)
        m = jnp.max(sm)
        i = jnp.min(jnp.where(sm == m, flat, BIG))
        found = m > NEG_INF
        i_safe = jnp.where(found, i, 0)
        r = i_safe // LANES
        l = i_safe % LANES

        def ext(ref):
            row = ref[b, pl.ds(r, 1), :]
            return jnp.max(jnp.where(lane == l, row, NEG_INF))

        bx1o = ext(x1o_ref)
        by1o = ext(y1o_ref)
        bx2o = ext(x2o_ref)
        by2o = ext(y2o_ref)
        bcls = ext(cls_ref)
        a2c = (bx2o - bx1o) * (by2o - by1o)

        ltx = jnp.maximum(sx1_ref[b], bx1o)
        lty = jnp.maximum(sy1_ref[b], by1o)
        rbx = jnp.minimum(sx2_ref[b], bx2o)
        rby = jnp.minimum(sy2_ref[b], by2o)
        w = jnp.maximum(rbx - ltx, 0.0)
        h = jnp.maximum(rby - lty, 0.0)
        inter = w * h
        iou = inter / (sa_ref[b] + a2c - inter + 1e-7)
        supped = jnp.max(jnp.where(iou > IOU_THRES, 1.0, 0.0)) > 0.0
        accept = found & (~supped) & (cnt < MAX_DET)

        off = bcls * MAX_WH
        det = jnp.where(
            lane6 == 0, bx1o - off,
            jnp.where(lane6 == 1, by1o - off,
                      jnp.where(lane6 == 2, bx2o - off,
                                jnp.where(lane6 == 3, by2o - off,
                                          jnp.where(lane6 == 4, m, bcls)))))

        @pl.when(accept)
        def _():
            o_ref[b, pl.ds(cnt, 1), :] = det
            onehot = (selrow == cnt // LANES) & (sellane == cnt % LANES)
            sx1_ref[b] = jnp.where(onehot, bx1o, sx1_ref[b])
            sy1_ref[b] = jnp.where(onehot, by1o, sy1_ref[b])
            sx2_ref[b] = jnp.where(onehot, bx2o, sx2_ref[b])
            sy2_ref[b] = jnp.where(onehot, by2o, sy2_ref[b])
            sa_ref[b] = jnp.where(onehot, a2c, sa_ref[b])

        cnt2 = cnt + jnp.where(accept, 1, 0)
        v2 = jnp.where(found, m, v)
        i2 = jnp.where(found, i, ilast)
        alive2 = alive & found & (cnt2 < MAX_DET)
        return (v2, i2, cnt2, alive2)

    def cond(states):
        a = states[0][3]
        for st in states[1:]:
            a = a | st[3]
        return a

    def body(states):
        return tuple(one_batch(b, st) for b, st in enumerate(states))

    init = tuple((jnp.float32(jnp.inf), jnp.int32(-1), jnp.int32(0),
                  jnp.bool_(True)) for _ in range(B))
    jax.lax.while_loop(cond, body, init)


@functools.partial(jax.jit, static_argnames=("interpret",))
def kernel(x, interpret=False):
    B = x.shape[0]
    xt = jnp.pad(x.transpose(0, 2, 1), ((0, 0), (0, 0), (0, NPAD - N)),
                 constant_values=-100.0)
    plane = jax.ShapeDtypeStruct((B, 1, NPAD), jnp.float32)
    planes = pl.pallas_call(
        _decode_body,
        grid=(B, NPAD // DEC_CHUNK),
        in_specs=[pl.BlockSpec((1, 85, DEC_CHUNK), lambda b, c: (b, 0, c))],
        out_specs=[pl.BlockSpec((1, 1, DEC_CHUNK), lambda b, c: (b, 0, c))] * 6,
        out_shape=[plane] * 6,
        interpret=interpret,
    )(xt)
    planes2d = [p.reshape(B, ROWS, LANES) for p in planes]
    dets = pl.pallas_call(
        _nms_body,
        in_specs=[pl.BlockSpec((B, ROWS, LANES), lambda: (0, 0, 0))] * 6,
        out_specs=pl.BlockSpec((B, MAX_DET, 6), lambda: (0, 0, 0)),
        out_shape=jax.ShapeDtypeStruct((B, MAX_DET, 6), jnp.float32),
        scratch_shapes=[pltpu.VMEM((B, SELR, LANES), jnp.float32)] * 5,
        interpret=interpret,
    )(*planes2d)
    return dets


# speedup vs baseline: 1.4934x; 1.4913x over previous
import functools

import jax
import jax.numpy as jnp
from jax.experimental import pallas as pl

CONF_THRES = 0.25
IOU_THRES = 0.45
MAX_DET = 300
MAX_WH = 4096.0
IMG = 640.0

N = 20000
NPAD = 20480
ROWS = 160
LANES = 128

NEG_INF = float("-inf")


DEC_CHUNK = 2048


def _decode_body(x_ref, s_ref, x1o_ref, y1o_ref, x2o_ref, y2o_ref,
                 x1_ref, y1_ref, x2_ref, y2_ref, cls_ref, a2_ref):
    xb = x_ref[0]
    cx = jax.nn.sigmoid(xb[0:1, :]) * IMG
    cy = jax.nn.sigmoid(xb[1:2, :]) * IMG
    hw = (jax.nn.sigmoid(xb[2:3, :]) * IMG) * 0.5
    hh = (jax.nn.sigmoid(xb[3:4, :]) * IMG) * 0.5
    obj = jax.nn.sigmoid(xb[4:5, :])
    cls_p = jax.nn.sigmoid(xb[5:85, :])
    conf = obj * cls_p
    score = jnp.max(conf, axis=0, keepdims=True)
    ii = jax.lax.broadcasted_iota(jnp.int32, conf.shape, 0)
    cls = jnp.min(jnp.where(conf == score, ii, 2**30),
                  axis=0, keepdims=True)
    clsf = cls.astype(jnp.float32)

    x1 = cx - hw
    y1 = cy - hh
    x2 = cx + hw
    y2 = cy + hh
    off = clsf * MAX_WH
    x1o = x1 + off
    y1o = y1 + off
    x2o = x2 + off
    y2o = y2 + off
    a2 = (x2o - x1o) * (y2o - y1o)
    s = jnp.where(score > CONF_THRES, score, NEG_INF)

    s_ref[0] = s
    x1o_ref[0] = x1o
    y1o_ref[0] = y1o
    x2o_ref[0] = x2o
    y2o_ref[0] = y2o
    x1_ref[0] = x1
    y1_ref[0] = y1
    x2_ref[0] = x2
    y2_ref[0] = y2
    cls_ref[0] = clsf
    a2_ref[0] = a2


def _nms_body(s_ref, x1o_ref, y1o_ref, x2o_ref, y2o_ref,
              x1_ref, y1_ref, x2_ref, y2_ref, cls_ref, a2_ref, o_ref):
    B = s_ref.shape[0]
    flat = (jax.lax.broadcasted_iota(jnp.int32, (ROWS, LANES), 0) * LANES
            + jax.lax.broadcasted_iota(jnp.int32, (ROWS, LANES), 1))
    lane = jax.lax.broadcasted_iota(jnp.int32, (1, LANES), 1)
    lane6 = jax.lax.broadcasted_iota(jnp.int32, (1, 6), 1)

    def one_batch(b, t, s):
        m = jnp.max(s)
        i = jnp.min(jnp.where(s == m, flat, 2**30))
        ok = m > NEG_INF
        r = i // LANES
        l = i % LANES

        def ext(ref):
            row = ref[b, pl.ds(r, 1), :]
            return jnp.max(jnp.where(lane == l, row, NEG_INF))

        bx1o = ext(x1o_ref)
        by1o = ext(y1o_ref)
        bx2o = ext(x2o_ref)
        by2o = ext(y2o_ref)
        bx1 = ext(x1_ref)
        by1 = ext(y1_ref)
        bx2 = ext(x2_ref)
        by2 = ext(y2_ref)
        bcls = ext(cls_ref)

        a1 = (bx2o - bx1o) * (by2o - by1o)
        ltx = jnp.maximum(bx1o, x1o_ref[b])
        lty = jnp.maximum(by1o, y1o_ref[b])
        rbx = jnp.minimum(bx2o, x2o_ref[b])
        rby = jnp.minimum(by2o, y2o_ref[b])
        w = jnp.maximum(rbx - ltx, 0.0)
        h = jnp.maximum(rby - lty, 0.0)
        inter = w * h
        iou = inter / (a1 + a2_ref[b] - inter + 1e-7)
        sup = (iou > IOU_THRES) | (flat == i)
        s_new = jnp.where(ok, jnp.where(sup, NEG_INF, s), s)

        okf = jnp.where(ok, 1.0, 0.0).astype(jnp.float32)
        msafe = jnp.where(ok, m, 0.0)
        det = jnp.where(
            lane6 == 0, bx1,
            jnp.where(lane6 == 1, by1,
                      jnp.where(lane6 == 2, bx2,
                                jnp.where(lane6 == 3, by2,
                                          jnp.where(lane6 == 4, msafe, bcls)))))
        o_ref[b, pl.ds(t, 1), :] = det * okf
        return s_new

    def step(t, ss):
        return tuple(one_batch(b, t, s) for b, s in enumerate(ss))

    jax.lax.fori_loop(0, MAX_DET, step,
                      tuple(s_ref[b] for b in range(B)))


@functools.partial(jax.jit, static_argnames=("interpret",))
def kernel(x, interpret=False):
    B = x.shape[0]
    xt = jnp.pad(x.transpose(0, 2, 1), ((0, 0), (0, 0), (0, NPAD - N)),
                 constant_values=-100.0)
    plane = jax.ShapeDtypeStruct((B, 1, NPAD), jnp.float32)
    planes = pl.pallas_call(
        _decode_body,
        grid=(B, NPAD // DEC_CHUNK),
        in_specs=[pl.BlockSpec((1, 85, DEC_CHUNK), lambda b, c: (b, 0, c))],
        out_specs=[pl.BlockSpec((1, 1, DEC_CHUNK), lambda b, c: (b, 0, c))] * 11,
        out_shape=[plane] * 11,
        interpret=interpret,
    )(xt)
    planes2d = [p.reshape(B, ROWS, LANES) for p in planes]
    dets = pl.pallas_call(
        _nms_body,
        in_specs=[pl.BlockSpec((B, ROWS, LANES), lambda: (0, 0, 0))] * 11,
        out_specs=pl.BlockSpec((B, MAX_DET, 6), lambda: (0, 0, 0)),
        out_shape=jax.ShapeDtypeStruct((B, MAX_DET, 6), jnp.float32),
        interpret=interpret,
    )(*planes2d)
    return dets
